# baseline (speedup 1.0000x reference)
import jax
import jax.numpy as jnp
from jax import lax
from jax.experimental import pallas as pl
from jax.experimental.pallas import tpu as pltpu
from jax.experimental.pallas import tpu_sc as plsc

_MAXL = 20
_D = 128
_TOTAL = 40960
_B = 4096
_NOUT = 8
_NN = 4
_NC = 2
_NS = 16
_NW = _NC * _NS
_BPW = _B // _NW
_G = 4
_NG = _BPW // _G
_SLOT = 24
_ROWS = _G * _SLOT
_DC = _D // 16
_WN = _MAXL * _D


def _body(va, vb, vc, vd, oa, ob, oc, od, w_hbm, bias_hbm, out_hbm,
          w_v, bias_v, off_a, off_b, off_c, off_d,
          idx0, idx1, rows0, rows1, acc_buf, out_v, sem0, sem1):
    vals = [va, vb, vc, vd]
    offs = [oa, ob, oc, od]
    off_refs = [off_a, off_b, off_c, off_d]
    idx_refs = [idx0, idx1]
    row_refs = [rows0, rows1]
    sems = [sem0, sem1]
    wid = lax.axis_index("s") * _NC + lax.axis_index("c")
    base = wid * _BPW

    pltpu.sync_copy(w_hbm, w_v)
    pltpu.sync_copy(bias_hbm, bias_v.at[pl.ds(0, _NOUT)])
    pltpu.sync_copy(bias_hbm, bias_v.at[pl.ds(_NOUT, _NOUT)])
    for n in range(_NN):
        pltpu.sync_copy(offs[n].at[pl.ds(base, _BPW + 1)],
                        off_refs[n].at[pl.ds(0, _BPW + 1)])

    iota = lax.iota(jnp.int32, 16)
    bias_vec = bias_v[...]

    def issue_gather(grp, n, buf):
        off_chunk = off_refs[n][pl.ds(grp * _G, 16)]
        for g in range(_G):
            s = off_chunk[g]
            c0 = jnp.minimum(s + iota, _TOTAL - 1)
            c1 = jnp.minimum(s + 8 + iota, _TOTAL - 1)
            idx_refs[buf][pl.ds(g * _SLOT, 16)] = c0
            idx_refs[buf][pl.ds(g * _SLOT + 8, 16)] = c1
        return pltpu.async_copy(vals[n].at[idx_refs[buf]], row_refs[buf],
                                sems[buf])

    issue_gather(0, 0, 0)

    def group_body(grp, carry):
        accs = [jnp.zeros((16,), jnp.float32)] * (_G * _NOUT)
        for n in range(_NN):
            buf = n & 1
            if n < _NN - 1:
                issue_gather(grp, n + 1, buf ^ 1)
            else:
                @pl.when(grp < _NG - 1)
                def _():
                    issue_gather(grp + 1, 0, buf ^ 1)
            pltpu.make_async_copy(vals[n].at[idx_refs[buf]], row_refs[buf],
                                  sems[buf]).wait()

            off_chunk = off_refs[n][pl.ds(grp * _G, 16)]
            lens = [off_chunk[g + 1] - off_chunk[g] for g in range(_G)]
            upper = lens[0]
            for g in range(1, _G):
                upper = jnp.maximum(upper, lens[g])
            upper = jnp.minimum(upper, _MAXL)
            rows_v = row_refs[buf]

            upper = jnp.bitwise_and(upper + 1, jnp.int32(~1))
            zero = jnp.zeros((16,), jnp.float32)
            for g in range(_G):
                def z_body(r, c, g=g, rows_v=rows_v):
                    for dc in range(_DC):
                        rows_v[g * _SLOT + r, pl.ds(dc * 16, 16)] = zero
                    return c
                lax.fori_loop(jnp.minimum(lens[g], upper), upper, z_body,
                              jnp.int32(0))

            def p_body(p0, acc_t, n=n, rows_v=rows_v):
                acc_l = list(acc_t)
                for u in range(2):
                    p = p0 + u
                    pbase = p * _D
                    for dcp in range(_DC // 2):
                        vch = []
                        for g in range(_G):
                            vch.append(rows_v[g * _SLOT + p,
                                              pl.ds(dcp * 32, 16)])
                            vch.append(rows_v[g * _SLOT + p,
                                              pl.ds(dcp * 32 + 16, 16)])
                        for o in range(_NOUT):
                            wi = w_v[pl.ds((o * (_NN * _WN) + n * _WN + pbase
                                            + dcp * 32) // 2, 16)]
                            wab = plsc.bitcast(wi, jnp.bfloat16)
                            w0, w1 = plsc.unpack(
                                wab, format=plsc.PackFormat.INTERLEAVED)
                            for g in range(_G):
                                acc_l[g * _NOUT + o] = (
                                    acc_l[g * _NOUT + o]
                                    + vch[2 * g] * w0 + vch[2 * g + 1] * w1)
                return tuple(acc_l)

            accs = list(plsc.parallel_loop(0, upper, step=2,
                                           carry=tuple(accs))(p_body))

        for i in range(_G * _NOUT):
            acc_buf[pl.ds(i * 16, 16)] = accs[i]
        for h in range(_G // 2):
            r = jnp.zeros((16,), jnp.float32)
            for l in range(16):
                r = r + plsc.load_gather(acc_buf, [h * 256 + iota * 16 + l])
            out_v[pl.ds(grp * (_G * _NOUT) + h * 16, 16)] = r + bias_vec
        return carry

    lax.fori_loop(0, _NG, group_body, jnp.int32(0))
    pltpu.sync_copy(out_v, out_hbm.at[pl.ds(base * _NOUT, _BPW * _NOUT)])


def kernel(values_a, values_b, values_c, values_d,
           offsets_a, offsets_b, offsets_c, offsets_d, W, b):
    mesh = plsc.VectorSubcoreMesh(core_axis_name="c", subcore_axis_name="s",
                                  num_cores=_NC, num_subcores=_NS)
    k = pl.kernel(
        _body,
        out_type=jax.ShapeDtypeStruct((_B * _NOUT,), jnp.float32),
        mesh=mesh,
        compiler_params=pltpu.CompilerParams(needs_layout_passes=False,
                                             disable_bounds_checks=True),
        scratch_types=[
            pltpu.VMEM((_NOUT * _NN * _WN // 2,), jnp.int32),
            pltpu.VMEM((16,), jnp.float32),
            pltpu.VMEM((_BPW + 16,), jnp.int32),
            pltpu.VMEM((_BPW + 16,), jnp.int32),
            pltpu.VMEM((_BPW + 16,), jnp.int32),
            pltpu.VMEM((_BPW + 16,), jnp.int32),
            pltpu.VMEM((_ROWS,), jnp.int32),
            pltpu.VMEM((_ROWS,), jnp.int32),
            pltpu.VMEM((_ROWS, _D), jnp.float32),
            pltpu.VMEM((_ROWS, _D), jnp.float32),
            pltpu.VMEM((_G * _NOUT * 16,), jnp.float32),
            pltpu.VMEM((_BPW * _NOUT,), jnp.float32),
            pltpu.SemaphoreType.DMA,
            pltpu.SemaphoreType.DMA,
        ],
    )
    w_pairs = (W.astype(jnp.bfloat16)
               .reshape(_NOUT, _NN, _MAXL, _DC // 2, 2, 16)
               .transpose(0, 1, 2, 3, 5, 4)
               .reshape(-1, 2))
    w_prep = jax.lax.bitcast_convert_type(w_pairs, jnp.int32).reshape(
        _NOUT * _NN * _WN // 2)
    out = k(values_a, values_b, values_c, values_d,
            offsets_a.astype(jnp.int32), offsets_b.astype(jnp.int32),
            offsets_c.astype(jnp.int32), offsets_d.astype(jnp.int32),
            w_prep, b)
    return out.reshape(_B, _NOUT)


# speedup vs baseline: 1.0959x; 1.0959x over previous
import jax
import jax.numpy as jnp
from jax import lax
from jax.experimental import pallas as pl
from jax.experimental.pallas import tpu as pltpu
from jax.experimental.pallas import tpu_sc as plsc

_MAXL = 20
_D = 128
_TOTAL = 40960
_B = 4096
_NOUT = 8
_NN = 4
_NC = 2
_NS = 16
_NW = _NC * _NS
_BPW = _B // _NW
_G = 4
_NG = _BPW // _G
_SLOT = 24
_ROWS = _G * _SLOT
_DC = _D // 16
_WN = _MAXL * _D


def _body(va, vb, vc, vd, oa, ob, oc, od, w_hbm, bias_hbm, out_hbm,
          w_v, bias_v, off_a, off_b, off_c, off_d,
          idx0, idx1, rows0, rows1, acc_buf, out_v, sem0, sem1):
    vals = [va, vb, vc, vd]
    offs = [oa, ob, oc, od]
    off_refs = [off_a, off_b, off_c, off_d]
    idx_refs = [idx0, idx1]
    row_refs = [rows0, rows1]
    sems = [sem0, sem1]
    wid = lax.axis_index("s") * _NC + lax.axis_index("c")
    base = wid * _BPW

    pltpu.sync_copy(w_hbm, w_v)
    pltpu.sync_copy(bias_hbm, bias_v.at[pl.ds(0, _NOUT)])
    pltpu.sync_copy(bias_hbm, bias_v.at[pl.ds(_NOUT, _NOUT)])
    for n in range(_NN):
        pltpu.sync_copy(offs[n].at[pl.ds(base, _BPW + 1)],
                        off_refs[n].at[pl.ds(0, _BPW + 1)])

    iota = lax.iota(jnp.int32, 16)
    bias_vec = bias_v[...]

    def issue_gather(grp, n, buf):
        off_chunk = off_refs[n][pl.ds(grp * _G, 16)]
        for g in range(_G):
            s = off_chunk[g]
            c0 = jnp.minimum(s + iota, _TOTAL - 1)
            c1 = jnp.minimum(s + 8 + iota, _TOTAL - 1)
            idx_refs[buf][pl.ds(g * _SLOT, 16)] = c0
            idx_refs[buf][pl.ds(g * _SLOT + 8, 16)] = c1
        return pltpu.async_copy(vals[n].at[idx_refs[buf]], row_refs[buf],
                                sems[buf])

    issue_gather(0, 0, 0)

    def group_body(grp, carry):
        accs = [jnp.zeros((16,), jnp.float32)] * (_G * _NOUT)
        for n in range(_NN):
            buf = n & 1
            if n < _NN - 1:
                issue_gather(grp, n + 1, buf ^ 1)
            else:
                @pl.when(grp < _NG - 1)
                def _():
                    issue_gather(grp + 1, 0, buf ^ 1)
            pltpu.make_async_copy(vals[n].at[idx_refs[buf]], row_refs[buf],
                                  sems[buf]).wait()

            off_chunk = off_refs[n][pl.ds(grp * _G, 16)]
            lens = [off_chunk[g + 1] - off_chunk[g] for g in range(_G)]
            upper = lens[0]
            for g in range(1, _G):
                upper = jnp.maximum(upper, lens[g])
            upper = jnp.minimum(upper, _MAXL)
            rows_v = row_refs[buf]

            upper = jnp.bitwise_and(upper + 1, jnp.int32(~1))
            zero = jnp.zeros((16,), jnp.float32)
            for g in range(_G):
                def z_body(r, c, g=g, rows_v=rows_v):
                    for dc in range(_DC):
                        rows_v[g * _SLOT + r, pl.ds(dc * 16, 16)] = zero
                    return c
                lax.fori_loop(jnp.minimum(lens[g], upper), upper, z_body,
                              jnp.int32(0))

            def p_body(p0, acc_t, n=n, rows_v=rows_v):
                acc_l = list(acc_t)
                for u in range(2):
                    p = p0 + u
                    pbase = p * _D
                    for dcp in range(_DC // 2):
                        vch = []
                        for g in range(_G):
                            vch.append(rows_v[g * _SLOT + p,
                                              pl.ds(dcp * 32, 16)])
                            vch.append(rows_v[g * _SLOT + p,
                                              pl.ds(dcp * 32 + 16, 16)])
                        for o in range(_NOUT):
                            wb = o * (_NN * _WN) + n * _WN + pbase + dcp * 32
                            w0 = w_v[pl.ds(wb, 16)]
                            w1 = w_v[pl.ds(wb + 16, 16)]
                            for g in range(_G):
                                acc_l[g * _NOUT + o] = (
                                    acc_l[g * _NOUT + o]
                                    + vch[2 * g] * w0 + vch[2 * g + 1] * w1)
                return tuple(acc_l)

            accs = list(plsc.parallel_loop(0, upper, step=2,
                                           carry=tuple(accs))(p_body))

        for i in range(_G * _NOUT):
            acc_buf[pl.ds(i * 16, 16)] = accs[i]
        for h in range(_G // 2):
            r = jnp.zeros((16,), jnp.float32)
            for l in range(16):
                r = r + plsc.load_gather(acc_buf, [h * 256 + iota * 16 + l])
            out_v[pl.ds(grp * (_G * _NOUT) + h * 16, 16)] = r + bias_vec
        return carry

    lax.fori_loop(0, _NG, group_body, jnp.int32(0))
    pltpu.sync_copy(out_v, out_hbm.at[pl.ds(base * _NOUT, _BPW * _NOUT)])


def kernel(values_a, values_b, values_c, values_d,
           offsets_a, offsets_b, offsets_c, offsets_d, W, b):
    mesh = plsc.VectorSubcoreMesh(core_axis_name="c", subcore_axis_name="s",
                                  num_cores=_NC, num_subcores=_NS)
    k = pl.kernel(
        _body,
        out_type=jax.ShapeDtypeStruct((_B * _NOUT,), jnp.float32),
        mesh=mesh,
        compiler_params=pltpu.CompilerParams(needs_layout_passes=False,
                                             disable_bounds_checks=True),
        scratch_types=[
            pltpu.VMEM((_NOUT * _NN * _WN,), jnp.float32),
            pltpu.VMEM((16,), jnp.float32),
            pltpu.VMEM((_BPW + 16,), jnp.int32),
            pltpu.VMEM((_BPW + 16,), jnp.int32),
            pltpu.VMEM((_BPW + 16,), jnp.int32),
            pltpu.VMEM((_BPW + 16,), jnp.int32),
            pltpu.VMEM((_ROWS,), jnp.int32),
            pltpu.VMEM((_ROWS,), jnp.int32),
            pltpu.VMEM((_ROWS, _D), jnp.float32),
            pltpu.VMEM((_ROWS, _D), jnp.float32),
            pltpu.VMEM((_G * _NOUT * 16,), jnp.float32),
            pltpu.VMEM((_BPW * _NOUT,), jnp.float32),
            pltpu.SemaphoreType.DMA,
            pltpu.SemaphoreType.DMA,
        ],
    )
    w_prep = W.reshape(_NOUT * _NN * _WN)
    out = k(values_a, values_b, values_c, values_d,
            offsets_a.astype(jnp.int32), offsets_b.astype(jnp.int32),
            offsets_c.astype(jnp.int32), offsets_d.astype(jnp.int32),
            w_prep, b)
    return out.reshape(_B, _NOUT)


# speedup vs baseline: 1.1668x; 1.0647x over previous
import jax
import jax.numpy as jnp
from jax import lax
from jax.experimental import pallas as pl
from jax.experimental.pallas import tpu as pltpu
from jax.experimental.pallas import tpu_sc as plsc

_MAXL = 20
_D = 128
_TOTAL = 40960
_B = 4096
_NOUT = 8
_NN = 4
_NC = 2
_NS = 16
_NW = _NC * _NS
_BPW = _B // _NW
_G = 4
_NG = _BPW // _G
_SLOT = 24
_ROWS = _G * _SLOT
_DC = _D // 16
_WN = _MAXL * _D


def _body(va, vb, vc, vd, oa, ob, oc, od, pa, pb, pc_, pd, w_hbm, bias_hbm,
          out_hbm, w_v, bias_v, off_a, off_b, off_c, off_d,
          perm_a, perm_b, perm_c, perm_d,
          idx0, idx1, rows0, rows1, acc_buf, out_v, sem0, sem1):
    vals = [va, vb, vc, vd]
    offs = [oa, ob, oc, od]
    perms = [pa, pb, pc_, pd]
    off_refs = [off_a, off_b, off_c, off_d]
    perm_refs = [perm_a, perm_b, perm_c, perm_d]
    idx_refs = [idx0, idx1]
    row_refs = [rows0, rows1]
    sems = [sem0, sem1]
    wid = lax.axis_index("s") * _NC + lax.axis_index("c")
    base = wid * _BPW

    pltpu.sync_copy(w_hbm, w_v)
    pltpu.sync_copy(bias_hbm, bias_v.at[pl.ds(0, _NOUT)])
    pltpu.sync_copy(bias_hbm, bias_v.at[pl.ds(_NOUT, _NOUT)])
    for n in range(_NN):
        pltpu.sync_copy(offs[n].at[pl.ds(base, _BPW + 1)],
                        off_refs[n].at[pl.ds(0, _BPW + 1)])
        pltpu.sync_copy(perms[n].at[pl.ds(base, _BPW)],
                        perm_refs[n].at[pl.ds(0, _BPW)])

    iota = lax.iota(jnp.int32, 16)
    iota_lo = jnp.bitwise_and(iota, 7)
    mask_lo = jnp.where(iota < _NOUT, jnp.float32(1.0), jnp.float32(0.0))
    bias_vec = bias_v[...]

    for i in range(_BPW * _NOUT // 16):
        out_v[pl.ds(i * 16, 16)] = bias_vec

    def local_ids(grp, n):
        pch = perm_refs[n][pl.ds(grp * _G, 16)]
        return jnp.bitwise_and(pch - base, _BPW - 1)

    def issue_gather(grp, n, buf):
        li = local_ids(grp, n)
        sv = plsc.load_gather(off_refs[n], [li])
        for g in range(_G):
            s = sv[g]
            c0 = jnp.minimum(s + iota, _TOTAL - 1)
            c1 = jnp.minimum(s + 8 + iota, _TOTAL - 1)
            idx_refs[buf][pl.ds(g * _SLOT, 16)] = c0
            idx_refs[buf][pl.ds(g * _SLOT + 8, 16)] = c1
        return pltpu.async_copy(vals[n].at[idx_refs[buf]], row_refs[buf],
                                sems[buf])

    issue_gather(0, 0, 0)

    def group_body(grp, carry):
        for n in range(_NN):
            buf = n & 1
            if n < _NN - 1:
                issue_gather(grp, n + 1, buf ^ 1)
            else:
                @pl.when(grp < _NG - 1)
                def _():
                    issue_gather(grp + 1, 0, buf ^ 1)
            pltpu.make_async_copy(vals[n].at[idx_refs[buf]], row_refs[buf],
                                  sems[buf]).wait()

            li = local_ids(grp, n)
            sv = plsc.load_gather(off_refs[n], [li])
            ev = plsc.load_gather(off_refs[n], [li + 1])
            lv = ev - sv
            lens = [lv[g] for g in range(_G)]
            upper = lens[0]
            for g in range(1, _G):
                upper = jnp.maximum(upper, lens[g])
            upper = jnp.minimum(upper, _MAXL)
            rows_v = row_refs[buf]

            upper = jnp.bitwise_and(upper + 1, jnp.int32(~1))
            zero = jnp.zeros((16,), jnp.float32)
            for g in range(_G):
                def z_body(r, c, g=g, rows_v=rows_v):
                    for dc in range(_DC):
                        rows_v[g * _SLOT + r, pl.ds(dc * 16, 16)] = zero
                    return c
                lax.fori_loop(jnp.minimum(lens[g], upper), upper, z_body,
                              jnp.int32(0))

            def p_body(p0, acc_t, n=n, rows_v=rows_v):
                acc_l = list(acc_t)
                for u in range(2):
                    p = p0 + u
                    pbase = p * _D
                    for dcp in range(_DC // 2):
                        vch = []
                        for g in range(_G):
                            vch.append(rows_v[g * _SLOT + p,
                                              pl.ds(dcp * 32, 16)])
                            vch.append(rows_v[g * _SLOT + p,
                                              pl.ds(dcp * 32 + 16, 16)])
                        for o in range(_NOUT):
                            wb = (o * (_NN * _WN) + n * _WN + pbase
                                  + dcp * 32)
                            w0 = w_v[pl.ds(wb, 16)]
                            w1 = w_v[pl.ds(wb + 16, 16)]
                            for g in range(_G):
                                acc_l[g * _NOUT + o] = (
                                    acc_l[g * _NOUT + o]
                                    + vch[2 * g] * w0 + vch[2 * g + 1] * w1)
                return tuple(acc_l)

            accs = [jnp.zeros((16,), jnp.float32)] * (_G * _NOUT)
            accs = list(plsc.parallel_loop(0, upper, step=2,
                                           carry=tuple(accs))(p_body))

            for i in range(_G * _NOUT):
                acc_buf[pl.ds(i * 16, 16)] = accs[i]
            for g in range(_G):
                r = jnp.zeros((16,), jnp.float32)
                for l in range(16):
                    r = r + plsc.load_gather(
                        acc_buf, [g * (_NOUT * 16) + iota_lo * 16 + l])
                tgt = li[g] * _NOUT
                v = out_v[pl.ds(tgt, 16)]
                out_v[pl.ds(tgt, 16)] = v + r * mask_lo
        return carry

    lax.fori_loop(0, _NG, group_body, jnp.int32(0))
    pltpu.sync_copy(out_v.at[pl.ds(0, _BPW * _NOUT)],
                    out_hbm.at[pl.ds(base * _NOUT, _BPW * _NOUT)])


def kernel(values_a, values_b, values_c, values_d,
           offsets_a, offsets_b, offsets_c, offsets_d, W, b):
    mesh = plsc.VectorSubcoreMesh(core_axis_name="c", subcore_axis_name="s",
                                  num_cores=_NC, num_subcores=_NS)
    k = pl.kernel(
        _body,
        out_type=jax.ShapeDtypeStruct((_B * _NOUT,), jnp.float32),
        mesh=mesh,
        compiler_params=pltpu.CompilerParams(needs_layout_passes=False,
                                             disable_bounds_checks=True),
        scratch_types=[
            pltpu.VMEM((_NOUT * _NN * _WN,), jnp.float32),
            pltpu.VMEM((16,), jnp.float32),
            pltpu.VMEM((_BPW + 16,), jnp.int32),
            pltpu.VMEM((_BPW + 16,), jnp.int32),
            pltpu.VMEM((_BPW + 16,), jnp.int32),
            pltpu.VMEM((_BPW + 16,), jnp.int32),
            pltpu.VMEM((_BPW + 16,), jnp.int32),
            pltpu.VMEM((_BPW + 16,), jnp.int32),
            pltpu.VMEM((_BPW + 16,), jnp.int32),
            pltpu.VMEM((_BPW + 16,), jnp.int32),
            pltpu.VMEM((_ROWS,), jnp.int32),
            pltpu.VMEM((_ROWS,), jnp.int32),
            pltpu.VMEM((_ROWS, _D), jnp.float32),
            pltpu.VMEM((_ROWS, _D), jnp.float32),
            pltpu.VMEM((_G * _NOUT * 16,), jnp.float32),
            pltpu.VMEM((_BPW * _NOUT + 16,), jnp.float32),
            pltpu.SemaphoreType.DMA,
            pltpu.SemaphoreType.DMA,
        ],
    )
    w_prep = W.reshape(_NOUT * _NN * _WN)

    def mk_perm(off):
        lens = off[1:] - off[:-1]
        order = jnp.argsort(lens.reshape(_NW, _BPW), axis=1)
        return (order.astype(jnp.int32)
                + (jnp.arange(_NW, dtype=jnp.int32) * _BPW)[:, None]
                ).reshape(-1)

    offs = [offsets_a.astype(jnp.int32), offsets_b.astype(jnp.int32),
            offsets_c.astype(jnp.int32), offsets_d.astype(jnp.int32)]
    out = k(values_a, values_b, values_c, values_d,
            offs[0], offs[1], offs[2], offs[3],
            mk_perm(offs[0]), mk_perm(offs[1]),
            mk_perm(offs[2]), mk_perm(offs[3]),
            w_prep, b)
    return out.reshape(_B, _NOUT)
